# baseline (speedup 1.0000x reference)
import jax
import jax.numpy as jnp
from jax import lax
from jax.experimental import pallas as pl
from jax.experimental.pallas import tpu as pltpu
from jax.experimental.pallas import tpu_sc as plsc

B = 16384
ED = 64
NT = 5
NC, NS = 2, 16
NW = NC * NS
ROWS_W = B // NW
BUFR = 256

GD = NT * 128
BBLK = 2048
NB = B // BBLK


def _sc_gather_kernel(idx_hbm, tu, ti, tg, tc, tt, out,
                      idx_v, rows_a, rows_b, sem, wsem_a, wsem_b):
    wid = lax.axis_index("s") * NC + lax.axis_index("c")
    base = wid * ROWS_W
    pltpu.sync_copy(idx_hbm.at[wid], idx_v)
    tbls = [tu, ti, tg, tc, tt]
    bufs = [rows_a, rows_b]
    wsems = [wsem_a, wsem_b]
    wcps = [None, None]
    nhalf = ROWS_W // BUFR
    for p in range(NT * nhalf):
        t, half = p // nhalf, p % nhalf
        buf = bufs[p % 2]
        if wcps[p % 2] is not None:
            wcps[p % 2].wait()
        off = half * BUFR

        @pl.loop(0, BUFR // 16)
        def _(g, t=t, buf=buf, off=off):
            v = idx_v[t, pl.ds(off + g * 16, 16)]
            for l in range(16):
                pltpu.async_copy(tbls[t].at[v[l]], buf.at[g * 16 + l], sem)

        pltpu.make_async_copy(tu.at[pl.ds(0, BUFR)], buf, sem).wait()
        wcps[p % 2] = pltpu.async_copy(
            buf,
            out.at[pl.ds(base + off, BUFR), pl.ds(t * 128, 128)],
            wsems[p % 2],
        )
    wcps[0].wait()
    wcps[1].wait()


def _sc_gather(idx_stack, tables):
    mesh = plsc.VectorSubcoreMesh(core_axis_name="c", subcore_axis_name="s")
    k = pl.kernel(
        _sc_gather_kernel,
        out_type=jax.ShapeDtypeStruct((B, GD), jnp.float32),
        mesh=mesh,
        scratch_types=[
            pltpu.VMEM((NT, ROWS_W), jnp.int32),
            pltpu.VMEM((BUFR, 128), jnp.float32),
            pltpu.VMEM((BUFR, 128), jnp.float32),
            pltpu.SemaphoreType.DMA,
            pltpu.SemaphoreType.DMA,
            pltpu.SemaphoreType.DMA,
        ],
    )
    return k(idx_stack, *tables)


def _mlp_body(g, par, w1, b1, w2, b2, w3, b3, out):
    gv = g[...]
    pv = par[...]

    def half(t):
        s = pv[:, t:t + 1]
        lo = gv[:, t * 128:t * 128 + ED]
        hi = gv[:, t * 128 + ED:(t + 1) * 128]
        return lo + s * (hi - lo)

    x = jnp.concatenate([half(t) for t in range(NT)], axis=1)
    h = jnp.dot(x, w1[...], preferred_element_type=jnp.float32,
                precision=lax.Precision.HIGHEST)
    h = jnp.maximum(h + b1[...], 0.0)
    h2 = jnp.dot(h, w2[...], preferred_element_type=jnp.float32,
                 precision=lax.Precision.HIGHEST)
    h2 = jnp.maximum(h2 + b2[...], 0.0)
    o = jnp.sum(h2 * w3[...], axis=1) + b3[0, 0]
    out[0, 0, :] = o


def _tc_mlp(g, par, W1, b1, W2, b2, W3, b3):
    full = lambda shape: pl.BlockSpec(shape, lambda i: (0, 0))
    out = pl.pallas_call(
        _mlp_body,
        grid=(NB,),
        in_specs=[
            pl.BlockSpec((BBLK, GD), lambda i: (i, 0)),
            pl.BlockSpec((BBLK, NT), lambda i: (i, 0)),
            full((NT * ED, ED)),
            full((1, ED)),
            full((ED, 32)),
            full((1, 32)),
            full((1, 32)),
            full((1, 1)),
        ],
        out_specs=pl.BlockSpec((1, 1, BBLK), lambda i: (i, 0, 0)),
        out_shape=jax.ShapeDtypeStruct((NB, 1, BBLK), jnp.float32),
    )(g, par, W1, b1.reshape(1, ED), W2, b2.reshape(1, 32),
      W3.reshape(1, 32), b3.reshape(1, 1))
    return out.reshape(-1)


def kernel(user, item, genre, country, tags,
           user_table, item_table, genre_table, country_table, tags_table,
           W1, b1, W2, b2, W3, b3):
    raw = [a.astype(jnp.int32) for a in (user, item, genre, country, tags)]
    idx = jnp.stack([a >> 1 for a in raw])
    idx = idx.reshape(NT, NW, ROWS_W).transpose(1, 0, 2)
    par = jnp.stack([a & 1 for a in raw], axis=1).astype(jnp.float32)
    g = _sc_gather(
        idx,
        (user_table.reshape(-1, 128), item_table.reshape(-1, 128),
         genre_table.reshape(-1, 128), country_table.reshape(-1, 128),
         tags_table.reshape(-1, 128)),
    )
    return _tc_mlp(g, par, W1, b1, W2, b2, W3, b3)


# speedup vs baseline: 1.5639x; 1.5639x over previous
import jax
import jax.numpy as jnp
from jax import lax
from jax.experimental import pallas as pl
from jax.experimental.pallas import tpu as pltpu
from jax.experimental.pallas import tpu_sc as plsc

B = 16384
ED = 64
NC, NS = 2, 16
NW = NC * NS
ROWS_W = B // NW
BUFR = 256

BBLK = 1024
NB = B // BBLK


def _make_sc_gather(nt):
    def body(idx_hbm, *refs):
        tbls = refs[:nt]
        outs = refs[nt:2 * nt]
        idx_v, rows_a, rows_b, sem, wsem_a, wsem_b = refs[2 * nt:]
        wid = lax.axis_index("s") * NC + lax.axis_index("c")
        base = wid * ROWS_W
        pltpu.sync_copy(idx_hbm.at[wid], idx_v)
        bufs = [rows_a, rows_b]
        wsems = [wsem_a, wsem_b]
        wcps = [None, None]
        nhalf = ROWS_W // BUFR
        for p in range(nt * nhalf):
            t, half = p // nhalf, p % nhalf
            buf = bufs[p % 2]
            if wcps[p % 2] is not None:
                wcps[p % 2].wait()
            off = half * BUFR

            @pl.loop(0, BUFR // 16)
            def _(g, t=t, buf=buf, off=off):
                v = idx_v[t, pl.ds(off + g * 16, 16)]
                for l in range(16):
                    pltpu.async_copy(
                        tbls[t].at[v[l]], buf.at[g * 16 + l], sem
                    )

            pltpu.make_async_copy(
                tbls[0].at[pl.ds(0, BUFR)], buf, sem
            ).wait()
            wcps[p % 2] = pltpu.async_copy(
                buf, outs[t].at[pl.ds(base + off, BUFR)], wsems[p % 2]
            )
        wcps[0].wait()
        if wcps[1] is not None:
            wcps[1].wait()

    mesh = plsc.VectorSubcoreMesh(core_axis_name="c", subcore_axis_name="s")
    return pl.kernel(
        body,
        out_type=tuple(
            jax.ShapeDtypeStruct((B, ED), jnp.float32) for _ in range(nt)
        ),
        mesh=mesh,
        scratch_types=[
            pltpu.VMEM((nt, ROWS_W), jnp.int32),
            pltpu.VMEM((BUFR, ED), jnp.float32),
            pltpu.VMEM((BUFR, ED), jnp.float32),
            pltpu.SemaphoreType.DMA,
            pltpu.SemaphoreType.DMA,
            pltpu.SemaphoreType.DMA,
        ],
    )


def _mlp_body(g0, g1, g2, g3, g4, w1, b1, w2, b2, w3, b3, out):
    x = jnp.concatenate(
        [g0[...], g1[...], g2[...], g3[...], g4[...]], axis=1
    )
    h = jnp.dot(x, w1[...], preferred_element_type=jnp.float32,
                precision=lax.Precision.HIGHEST)
    h = jnp.maximum(h + b1[...], 0.0)
    h2 = jnp.dot(h, w2[...], preferred_element_type=jnp.float32,
                 precision=lax.Precision.HIGHEST)
    h2 = jnp.maximum(h2 + b2[...], 0.0)
    o = jnp.sum(h2 * w3[...], axis=1) + b3[0, 0]
    out[0, 0, :] = o


def _tc_mlp(gs, W1, b1, W2, b2, W3, b3):
    full = lambda shape: pl.BlockSpec(shape, lambda i: (0, 0))
    out = pl.pallas_call(
        _mlp_body,
        grid=(NB,),
        in_specs=[pl.BlockSpec((BBLK, ED), lambda i: (i, 0))] * 5 + [
            full((5 * ED, ED)),
            full((1, ED)),
            full((ED, 32)),
            full((1, 32)),
            full((1, 32)),
            full((1, 1)),
        ],
        out_specs=pl.BlockSpec((1, 1, BBLK), lambda i: (i, 0, 0)),
        out_shape=jax.ShapeDtypeStruct((NB, 1, BBLK), jnp.float32),
    )(*gs, W1, b1.reshape(1, ED), W2, b2.reshape(1, 32),
      W3.reshape(1, 32), b3.reshape(1, 1))
    return out.reshape(-1)


def _widx(*cols):
    idx = jnp.stack([c.astype(jnp.int32) for c in cols])
    return idx.reshape(len(cols), NW, ROWS_W).transpose(1, 0, 2)


def kernel(user, item, genre, country, tags,
           user_table, item_table, genre_table, country_table, tags_table,
           W1, b1, W2, b2, W3, b3):
    g_item, g_genre, g_country, g_tags = _make_sc_gather(4)(
        _widx(item, genre, country, tags),
        item_table, genre_table, country_table, tags_table,
    )
    (g_user,) = _make_sc_gather(1)(_widx(user), user_table)
    return _tc_mlp(
        (g_user, g_item, g_genre, g_country, g_tags),
        W1, b1, W2, b2, W3, b3,
    )
